# Initial kernel scaffold; baseline (speedup 1.0000x reference)
import functools

import jax
import jax.numpy as jnp
from jax import lax
from jax.experimental import pallas as pl
from jax.experimental.pallas import tpu as pltpu
from jax.experimental.pallas import tpu_sc as plsc

N = 10000
N_PAD = 10240
E = 160000
E_PAD = 163840
NB = 80
EB = E_PAD // 16
CW = 128
RT = N_PAD // 16
DB = NB // 2
RBLK = 1024

_MESH = dict(core_axis_name="c", subcore_axis_name="s")


@functools.partial(
    pl.kernel,
    mesh=plsc.VectorSubcoreMesh(**_MESH),
    out_type=jax.ShapeDtypeStruct((2, N_PAD, 16), jnp.float32),
    scratch_types=[
        pltpu.VMEM((DB, 128), jnp.int32),
        pltpu.VMEM((128, 16), jnp.float32),
        pltpu.VMEM_SHARED((N_PAD, 16), jnp.float32),
    ],
)
def _deg_kernel(dst_hbm, ones_hbm, zeros_hbm, out_hbm, dst_v, ones_v, acc):
    cid = lax.axis_index("c")
    sid = lax.axis_index("s")
    pltpu.sync_copy(dst_hbm.at[sid, pl.ds(cid * DB, DB)], dst_v)
    pltpu.sync_copy(ones_hbm, ones_v)
    pltpu.sync_copy(zeros_hbm.at[pl.ds(sid * RT, RT)], acc.at[pl.ds(sid * RT, RT)])
    plsc.subcore_barrier()

    def body(j, carry):
        pltpu.sync_copy(ones_v, acc.at[dst_v.at[j]], add=True)
        return carry

    lax.fori_loop(0, DB, body, 0)
    plsc.subcore_barrier()
    pltpu.sync_copy(acc.at[pl.ds(sid * RT, RT)], out_hbm.at[cid, pl.ds(sid * RT, RT)])


def _make_scatter(nch):
    rpc = nch // 2

    @functools.partial(
        pl.kernel,
        mesh=plsc.VectorSubcoreMesh(**_MESH),
        out_type=jax.ShapeDtypeStruct((nch, N_PAD, CW), jnp.float32),
        scratch_types=[
            pltpu.VMEM((NB, 128), jnp.int32),
            pltpu.VMEM((NB, 128), jnp.int32),
            pltpu.VMEM((128, CW), jnp.float32),
            pltpu.VMEM_SHARED((N_PAD, CW), jnp.float32),
            pltpu.SemaphoreType.DMA,
        ],
    )
    def scat(g_hbm, src_hbm, dst_hbm, out_hbm, src_v, dst_v, buf, acc, sem):
        cid = lax.axis_index("c")
        sid = lax.axis_index("s")
        pltpu.sync_copy(dst_hbm.at[sid], dst_v)
        for rep in range(rpc):
            chunk = cid * rpc + rep
            pltpu.sync_copy(src_hbm.at[chunk, sid], src_v)
            pltpu.sync_copy(
                g_hbm.at[pl.ds(chunk * N_PAD + sid * RT, RT)],
                acc.at[pl.ds(sid * RT, RT)],
            )
            plsc.subcore_barrier()

            def body(j, carry):
                pltpu.async_copy(g_hbm.at[src_v.at[j]], buf, sem).wait()
                pltpu.sync_copy(buf, acc.at[dst_v.at[j]], add=True)
                return carry

            lax.fori_loop(0, NB, body, 0)
            plsc.subcore_barrier()
            pltpu.sync_copy(
                acc.at[pl.ds(sid * RT, RT)],
                out_hbm.at[chunk, pl.ds(sid * RT, RT)],
            )
            if rep + 1 < rpc:
                plsc.subcore_barrier()

    return scat


_scatter4 = _make_scatter(4)
_scatter2 = _make_scatter(2)


def _dot(a, b):
    return lax.dot_general(a, b, (((1,), (0,)), ((), ())),
                           preferred_element_type=jnp.float32)


def _mm1_body(x_ref, w_ref, d_ref, o_ref):
    g = _dot(x_ref[...], w_ref[...]) * d_ref[...]
    for k in range(4):
        o_ref[k] = g[:, k * CW:(k + 1) * CW]


def _mm1(x_pad, W1, dvec):
    return pl.pallas_call(
        _mm1_body,
        grid=(N_PAD // RBLK,),
        in_specs=[
            pl.BlockSpec((RBLK, 256), lambda i: (i, 0)),
            pl.BlockSpec((256, 512), lambda i: (0, 0)),
            pl.BlockSpec((RBLK, 1), lambda i: (i, 0)),
        ],
        out_specs=pl.BlockSpec((4, RBLK, CW), lambda i: (0, i, 0)),
        out_shape=jax.ShapeDtypeStruct((4, N_PAD, CW), jnp.float32),
    )(x_pad, W1, dvec)


def _mm2_body(s_ref, w_ref, b_ref, d_ref, o_ref):
    t = jnp.concatenate([s_ref[k] for k in range(4)], axis=1)
    t = jnp.maximum(t * d_ref[...] + b_ref[...], 0.0)
    g = _dot(t, w_ref[...]) * d_ref[...]
    o_ref[0] = g[:, :CW]
    o_ref[1] = g[:, CW:]


def _mm2(s1, W2, b1, dvec):
    return pl.pallas_call(
        _mm2_body,
        grid=(N_PAD // RBLK,),
        in_specs=[
            pl.BlockSpec((4, RBLK, CW), lambda i: (0, i, 0)),
            pl.BlockSpec((512, 256), lambda i: (0, 0)),
            pl.BlockSpec((1, 512), lambda i: (0, 0)),
            pl.BlockSpec((RBLK, 1), lambda i: (i, 0)),
        ],
        out_specs=pl.BlockSpec((2, RBLK, CW), lambda i: (0, i, 0)),
        out_shape=jax.ShapeDtypeStruct((2, N_PAD, CW), jnp.float32),
    )(s1, W2, b1, dvec)


def _fin_body(s_ref, b_ref, d_ref, o_ref):
    t = jnp.concatenate([s_ref[0], s_ref[1]], axis=1)
    o_ref[...] = t * d_ref[...] + b_ref[...]


def _fin(s2, b2, dvec):
    return pl.pallas_call(
        _fin_body,
        grid=(N_PAD // RBLK,),
        in_specs=[
            pl.BlockSpec((2, RBLK, CW), lambda i: (0, i, 0)),
            pl.BlockSpec((1, 256), lambda i: (0, 0)),
            pl.BlockSpec((RBLK, 1), lambda i: (i, 0)),
        ],
        out_specs=pl.BlockSpec((RBLK, 256), lambda i: (i, 0)),
        out_shape=jax.ShapeDtypeStruct((N_PAD, 256), jnp.float32),
    )(s2, b2, dvec)


def kernel(x, edge_index, W1, b1, W2, b2):
    src = edge_index[0].astype(jnp.int32)
    dst = edge_index[1].astype(jnp.int32)
    pad = N + (jnp.arange(E_PAD - E, dtype=jnp.int32) % (N_PAD - N))
    src_p = jnp.concatenate([src, pad]).reshape(16, EB)
    dst_t = jnp.concatenate([dst, pad]).reshape(16, NB, 128)
    src4 = (src_p[None] + (jnp.arange(4, dtype=jnp.int32) * N_PAD)[:, None, None]
            ).reshape(4, 16, NB, 128)
    src2 = src4[:2].reshape(2, 16, NB, 128)

    ones16 = jnp.ones((128, 16), jnp.float32)
    zeros16 = jnp.zeros((N_PAD, 16), jnp.float32)
    hist2 = _deg_kernel(dst_t, ones16, zeros16)
    hist = hist2[0, :, 0] + hist2[1, :, 0]
    dvec = jnp.where(jnp.arange(N_PAD) < N,
                     lax.rsqrt(hist + 1.0), 0.0).astype(jnp.float32)[:, None]

    x_pad = jnp.pad(x, ((0, N_PAD - N), (0, 0)))
    g1 = _mm1(x_pad, W1, dvec)
    s1 = _scatter4(g1.reshape(4 * N_PAD, CW), src4, dst_t)
    g2 = _mm2(s1, W2, b1.reshape(1, 512), dvec)
    s2 = _scatter2(g2.reshape(2 * N_PAD, CW), src2, dst_t)
    out = _fin(s2, b2.reshape(1, 256), dvec)
    return out[:N]


# speedup vs baseline: 10.9990x; 10.9990x over previous
import functools

import jax
import jax.numpy as jnp
from jax import lax
from jax.experimental import pallas as pl
from jax.experimental.pallas import tpu as pltpu
from jax.experimental.pallas import tpu_sc as plsc

N = 10000
N_PAD = 10240
E = 160000
E_PAD = 163840
NB = 80
EB = E_PAD // 16
CW = 128
RT = N_PAD // 16
DB = NB // 2
RBLK = 1024

_MESH = dict(core_axis_name="c", subcore_axis_name="s")


@functools.partial(
    pl.kernel,
    mesh=plsc.VectorSubcoreMesh(**_MESH),
    out_type=jax.ShapeDtypeStruct((2, N_PAD, 128), jnp.float32),
    scratch_types=[
        pltpu.VMEM((DB, 128), jnp.int32),
        pltpu.VMEM((128, 128), jnp.float32),
        pltpu.VMEM_SHARED((N_PAD, 128), jnp.float32),
    ],
)
def _deg_kernel(dst_hbm, ones_hbm, out_hbm, dst_v, ones_v, acc):
    cid = lax.axis_index("c")
    sid = lax.axis_index("s")
    pltpu.sync_copy(dst_hbm.at[sid, cid], dst_v)
    pltpu.sync_copy(ones_hbm.at[pl.ds(0, 128)], ones_v)
    pltpu.sync_copy(ones_hbm.at[pl.ds(sid * RT, RT)], acc.at[pl.ds(sid * RT, RT)])
    plsc.subcore_barrier()

    def body(j, carry):
        pltpu.sync_copy(ones_v, acc.at[dst_v.at[j]], add=True)
        return carry

    lax.fori_loop(0, DB, body, 0)
    plsc.subcore_barrier()
    pltpu.sync_copy(acc.at[pl.ds(sid * RT, RT)], out_hbm.at[cid, pl.ds(sid * RT, RT)])


def _make_scatter(nch):
    rpc = nch // 2

    @functools.partial(
        pl.kernel,
        mesh=plsc.VectorSubcoreMesh(**_MESH),
        out_type=jax.ShapeDtypeStruct((nch, N_PAD, CW), jnp.float32),
        scratch_types=[
            pltpu.VMEM((NB, 128), jnp.int32),
            pltpu.VMEM((NB, 128), jnp.int32),
            pltpu.VMEM((128, CW), jnp.float32),
            pltpu.VMEM_SHARED((N_PAD, CW), jnp.float32),
            pltpu.SemaphoreType.DMA,
        ],
    )
    def scat(g_hbm, src_hbm, dst_hbm, out_hbm, src_v, dst_v, buf, acc, sem):
        cid = lax.axis_index("c")
        sid = lax.axis_index("s")
        pltpu.sync_copy(dst_hbm.at[sid], dst_v)
        for rep in range(rpc):
            chunk = cid * rpc + rep
            pltpu.sync_copy(src_hbm.at[chunk, sid], src_v)
            pltpu.sync_copy(
                g_hbm.at[pl.ds(chunk * N_PAD + sid * RT, RT)],
                acc.at[pl.ds(sid * RT, RT)],
            )
            plsc.subcore_barrier()

            def body(j, carry):
                pltpu.async_copy(g_hbm.at[src_v.at[j]], buf, sem).wait()
                pltpu.sync_copy(buf, acc.at[dst_v.at[j]], add=True)
                return carry

            lax.fori_loop(0, NB, body, 0)
            plsc.subcore_barrier()
            pltpu.sync_copy(
                acc.at[pl.ds(sid * RT, RT)],
                out_hbm.at[chunk, pl.ds(sid * RT, RT)],
            )
            if rep + 1 < rpc:
                plsc.subcore_barrier()

    return scat


_scatter4 = _make_scatter(4)
_scatter2 = _make_scatter(2)


def _dot(a, b):
    return lax.dot_general(a, b, (((1,), (0,)), ((), ())),
                           preferred_element_type=jnp.float32)


def _mm1_body(x_ref, w_ref, d_ref, o_ref):
    g = _dot(x_ref[...], w_ref[...]) * d_ref[...]
    for k in range(4):
        o_ref[k] = g[:, k * CW:(k + 1) * CW]


def _mm1(x_pad, W1, dvec):
    return pl.pallas_call(
        _mm1_body,
        grid=(N_PAD // RBLK,),
        in_specs=[
            pl.BlockSpec((RBLK, 256), lambda i: (i, 0)),
            pl.BlockSpec((256, 512), lambda i: (0, 0)),
            pl.BlockSpec((RBLK, 1), lambda i: (i, 0)),
        ],
        out_specs=pl.BlockSpec((4, RBLK, CW), lambda i: (0, i, 0)),
        out_shape=jax.ShapeDtypeStruct((4, N_PAD, CW), jnp.float32),
    )(x_pad, W1, dvec)


def _mm2_body(s_ref, w_ref, b_ref, d_ref, o_ref):
    t = jnp.concatenate([s_ref[k] for k in range(4)], axis=1)
    t = jnp.maximum(t * d_ref[...] + b_ref[...], 0.0)
    g = _dot(t, w_ref[...]) * d_ref[...]
    o_ref[0] = g[:, :CW]
    o_ref[1] = g[:, CW:]


def _mm2(s1, W2, b1, dvec):
    return pl.pallas_call(
        _mm2_body,
        grid=(N_PAD // RBLK,),
        in_specs=[
            pl.BlockSpec((4, RBLK, CW), lambda i: (0, i, 0)),
            pl.BlockSpec((512, 256), lambda i: (0, 0)),
            pl.BlockSpec((1, 512), lambda i: (0, 0)),
            pl.BlockSpec((RBLK, 1), lambda i: (i, 0)),
        ],
        out_specs=pl.BlockSpec((2, RBLK, CW), lambda i: (0, i, 0)),
        out_shape=jax.ShapeDtypeStruct((2, N_PAD, CW), jnp.float32),
    )(s1, W2, b1, dvec)


def _fin_body(s_ref, b_ref, d_ref, o_ref):
    t = jnp.concatenate([s_ref[0], s_ref[1]], axis=1)
    o_ref[...] = t * d_ref[...] + b_ref[...]


def _fin(s2, b2, dvec):
    return pl.pallas_call(
        _fin_body,
        grid=(N_PAD // RBLK,),
        in_specs=[
            pl.BlockSpec((2, RBLK, CW), lambda i: (0, i, 0)),
            pl.BlockSpec((1, 256), lambda i: (0, 0)),
            pl.BlockSpec((RBLK, 1), lambda i: (i, 0)),
        ],
        out_specs=pl.BlockSpec((RBLK, 256), lambda i: (i, 0)),
        out_shape=jax.ShapeDtypeStruct((N_PAD, 256), jnp.float32),
    )(s2, b2, dvec)


def kernel(x, edge_index, W1, b1, W2, b2):
    src = edge_index[0].astype(jnp.int32)
    dst = edge_index[1].astype(jnp.int32)
    pad = N + (jnp.arange(E_PAD - E, dtype=jnp.int32) % (N_PAD - N))
    src_p = jnp.concatenate([src, pad]).reshape(16, EB)
    dst_t = jnp.concatenate([dst, pad]).reshape(16, NB, 128)
    src4 = (src_p[None] + (jnp.arange(4, dtype=jnp.int32) * N_PAD)[:, None, None]
            ).reshape(4, 16, NB, 128)
    src2 = src4[:2].reshape(2, 16, NB, 128)

    ones_pl = jnp.ones((N_PAD, 128), jnp.float32)
    hist2 = _deg_kernel(dst_t.reshape(16, 2, DB, 128), ones_pl)
    hist = hist2[0, :, 0] + hist2[1, :, 0] - 2.0
    dvec = jnp.where(jnp.arange(N_PAD) < N,
                     lax.rsqrt(hist + 1.0), 0.0).astype(jnp.float32)[:, None]

    x_pad = jnp.pad(x, ((0, N_PAD - N), (0, 0)))
    g1 = _mm1(x_pad, W1, dvec)
    s1 = _scatter4(g1.reshape(4 * N_PAD, CW), src4, dst_t)
    g2 = _mm2(s1, W2, b1.reshape(1, 512), dvec)
    s2 = _scatter2(g2.reshape(2 * N_PAD, CW), src2, dst_t)
    out = _fin(s2, b2.reshape(1, 256), dvec)
    return out[:N]


# speedup vs baseline: 13.6484x; 1.2409x over previous
import functools

import jax
import jax.numpy as jnp
from jax import lax
from jax.experimental import pallas as pl
from jax.experimental.pallas import tpu as pltpu
from jax.experimental.pallas import tpu_sc as plsc

N = 10000
N_PAD = 10240
E = 160000
E_PAD = 163840
NB = 80
EB = E_PAD // 16
CW = 128
RT = N_PAD // 16
DB = NB // 2
RBLK = 1024

_MESH = dict(core_axis_name="c", subcore_axis_name="s")


@functools.partial(
    pl.kernel,
    mesh=plsc.VectorSubcoreMesh(**_MESH),
    out_type=jax.ShapeDtypeStruct((2, N_PAD, 128), jnp.float32),
    scratch_types=[
        pltpu.VMEM((DB, 128), jnp.int32),
        pltpu.VMEM((128, 128), jnp.float32),
        pltpu.VMEM_SHARED((N_PAD, 128), jnp.float32),
    ],
)
def _deg_kernel(dst_hbm, ones_hbm, out_hbm, dst_v, ones_v, acc):
    cid = lax.axis_index("c")
    sid = lax.axis_index("s")
    pltpu.sync_copy(dst_hbm.at[sid, cid], dst_v)
    pltpu.sync_copy(ones_hbm.at[pl.ds(0, 128)], ones_v)
    pltpu.sync_copy(ones_hbm.at[pl.ds(sid * RT, RT)], acc.at[pl.ds(sid * RT, RT)])
    plsc.subcore_barrier()

    def body(j, carry):
        pltpu.sync_copy(ones_v, acc.at[dst_v.at[j]], add=True)
        return carry

    lax.fori_loop(0, DB, body, 0)
    plsc.subcore_barrier()
    pltpu.sync_copy(acc.at[pl.ds(sid * RT, RT)], out_hbm.at[cid, pl.ds(sid * RT, RT)])


def _make_scatter(nch):
    rpc = nch // 2

    @functools.partial(
        pl.kernel,
        mesh=plsc.VectorSubcoreMesh(**_MESH),
        out_type=jax.ShapeDtypeStruct((nch, N_PAD, CW), jnp.float32),
        scratch_types=[
            pltpu.VMEM((NB // 2, 128), jnp.int32),
            pltpu.VMEM((NB // 2, 128), jnp.int32),
            pltpu.VMEM((2, 128, CW), jnp.float32),
            pltpu.VMEM_SHARED((N_PAD, CW), jnp.float32),
            pltpu.SemaphoreType.DMA,
            pltpu.SemaphoreType.DMA,
        ],
    )
    def scat(g_hbm, src_hbm, dst_hbm, out_hbm, src_v, dst_v, buf, acc, s0, s1):
        cid = lax.axis_index("c")
        sid = lax.axis_index("s")
        NBH = NB // 2
        for rep in range(rpc):
            chunk = cid * rpc + rep
            pltpu.sync_copy(
                g_hbm.at[pl.ds(chunk * N_PAD + sid * RT, RT)],
                acc.at[pl.ds(sid * RT, RT)],
            )
            plsc.subcore_barrier()

            for half in range(2):
                pltpu.sync_copy(
                    src_hbm.at[chunk, sid, pl.ds(half * NBH, NBH)], src_v)
                pltpu.sync_copy(
                    dst_hbm.at[sid, pl.ds(half * NBH, NBH)], dst_v)

                pltpu.async_copy(g_hbm.at[src_v.at[0]], buf.at[0], s0)

                def body(h, carry):
                    j = 2 * h
                    pltpu.make_async_copy(
                        g_hbm.at[src_v.at[0]], buf.at[0], s0).wait()
                    pltpu.async_copy(g_hbm.at[src_v.at[j + 1]], buf.at[1], s1)
                    pltpu.sync_copy(buf.at[0], acc.at[dst_v.at[j]], add=True)
                    pltpu.make_async_copy(
                        g_hbm.at[src_v.at[0]], buf.at[1], s1).wait()
                    pltpu.async_copy(
                        g_hbm.at[src_v.at[lax.rem(j + 2, NBH)]], buf.at[0], s0)
                    pltpu.sync_copy(buf.at[1], acc.at[dst_v.at[j + 1]], add=True)
                    return carry

                lax.fori_loop(0, NBH // 2, body, 0)
                pltpu.make_async_copy(g_hbm.at[src_v.at[0]], buf.at[0], s0).wait()
            plsc.subcore_barrier()
            pltpu.sync_copy(
                acc.at[pl.ds(sid * RT, RT)],
                out_hbm.at[chunk, pl.ds(sid * RT, RT)],
            )
            if rep + 1 < rpc:
                plsc.subcore_barrier()

    return scat


_scatter4 = _make_scatter(4)
_scatter2 = _make_scatter(2)


def _dot(a, b):
    return lax.dot_general(a, b, (((1,), (0,)), ((), ())),
                           preferred_element_type=jnp.float32)


def _mm1_body(x_ref, w_ref, d_ref, o_ref):
    g = _dot(x_ref[...], w_ref[...]) * d_ref[...]
    for k in range(4):
        o_ref[k] = g[:, k * CW:(k + 1) * CW]


def _mm1(x_pad, W1, dvec):
    return pl.pallas_call(
        _mm1_body,
        grid=(N_PAD // RBLK,),
        in_specs=[
            pl.BlockSpec((RBLK, 256), lambda i: (i, 0)),
            pl.BlockSpec((256, 512), lambda i: (0, 0)),
            pl.BlockSpec((RBLK, 1), lambda i: (i, 0)),
        ],
        out_specs=pl.BlockSpec((4, RBLK, CW), lambda i: (0, i, 0)),
        out_shape=jax.ShapeDtypeStruct((4, N_PAD, CW), jnp.float32),
    )(x_pad, W1, dvec)


def _mm2_body(s_ref, w_ref, b_ref, d_ref, o_ref):
    t = jnp.concatenate([s_ref[k] for k in range(4)], axis=1)
    t = jnp.maximum(t * d_ref[...] + b_ref[...], 0.0)
    g = _dot(t, w_ref[...]) * d_ref[...]
    o_ref[0] = g[:, :CW]
    o_ref[1] = g[:, CW:]


def _mm2(s1, W2, b1, dvec):
    return pl.pallas_call(
        _mm2_body,
        grid=(N_PAD // RBLK,),
        in_specs=[
            pl.BlockSpec((4, RBLK, CW), lambda i: (0, i, 0)),
            pl.BlockSpec((512, 256), lambda i: (0, 0)),
            pl.BlockSpec((1, 512), lambda i: (0, 0)),
            pl.BlockSpec((RBLK, 1), lambda i: (i, 0)),
        ],
        out_specs=pl.BlockSpec((2, RBLK, CW), lambda i: (0, i, 0)),
        out_shape=jax.ShapeDtypeStruct((2, N_PAD, CW), jnp.float32),
    )(s1, W2, b1, dvec)


def _fin_body(s_ref, b_ref, d_ref, o_ref):
    t = jnp.concatenate([s_ref[0], s_ref[1]], axis=1)
    o_ref[...] = t * d_ref[...] + b_ref[...]


def _fin(s2, b2, dvec):
    return pl.pallas_call(
        _fin_body,
        grid=(N_PAD // RBLK,),
        in_specs=[
            pl.BlockSpec((2, RBLK, CW), lambda i: (0, i, 0)),
            pl.BlockSpec((1, 256), lambda i: (0, 0)),
            pl.BlockSpec((RBLK, 1), lambda i: (i, 0)),
        ],
        out_specs=pl.BlockSpec((RBLK, 256), lambda i: (i, 0)),
        out_shape=jax.ShapeDtypeStruct((N_PAD, 256), jnp.float32),
    )(s2, b2, dvec)


def kernel(x, edge_index, W1, b1, W2, b2):
    src = edge_index[0].astype(jnp.int32)
    dst = edge_index[1].astype(jnp.int32)
    pad = N + (jnp.arange(E_PAD - E, dtype=jnp.int32) % (N_PAD - N))
    src_p = jnp.concatenate([src, pad]).reshape(16, EB)
    dst_t = jnp.concatenate([dst, pad]).reshape(16, NB, 128)
    src4 = (src_p[None] + (jnp.arange(4, dtype=jnp.int32) * N_PAD)[:, None, None]
            ).reshape(4, 16, NB, 128)
    src2 = src4[:2].reshape(2, 16, NB, 128)

    ones_pl = jnp.ones((N_PAD, 128), jnp.float32)
    hist2 = _deg_kernel(dst_t.reshape(16, 2, DB, 128), ones_pl)
    hist = hist2[0, :, 0] + hist2[1, :, 0] - 2.0
    dvec = jnp.where(jnp.arange(N_PAD) < N,
                     lax.rsqrt(hist + 1.0), 0.0).astype(jnp.float32)[:, None]

    x_pad = jnp.pad(x, ((0, N_PAD - N), (0, 0)))
    g1 = _mm1(x_pad, W1, dvec)
    s1 = _scatter4(g1.reshape(4 * N_PAD, CW), src4, dst_t)
    g2 = _mm2(s1, W2, b1.reshape(1, 512), dvec)
    s2 = _scatter2(g2.reshape(2 * N_PAD, CW), src2, dst_t)
    out = _fin(s2, b2.reshape(1, 256), dvec)
    return out[:N]


# speedup vs baseline: 18.2507x; 1.3372x over previous
import functools

import jax
import jax.numpy as jnp
from jax import lax
from jax.experimental import pallas as pl
from jax.experimental.pallas import tpu as pltpu
from jax.experimental.pallas import tpu_sc as plsc

N = 10000
N_PAD = 10240
E = 160000
E_PAD = 163840
NB = 80
EB = E_PAD // 16
CW = 128
RT = N_PAD // 16
DB = NB // 2
RBLK = 1024

_MESH = dict(core_axis_name="c", subcore_axis_name="s")


@functools.partial(
    pl.kernel,
    mesh=plsc.VectorSubcoreMesh(**_MESH),
    out_type=jax.ShapeDtypeStruct((2, N_PAD, 128), jnp.float32),
    scratch_types=[
        pltpu.VMEM((DB, 128), jnp.int32),
        pltpu.VMEM((128, 128), jnp.float32),
        pltpu.VMEM_SHARED((N_PAD, 128), jnp.float32),
    ],
)
def _deg_kernel(dst_hbm, ones_hbm, out_hbm, dst_v, ones_v, acc):
    cid = lax.axis_index("c")
    sid = lax.axis_index("s")
    pltpu.sync_copy(dst_hbm.at[sid, cid], dst_v)
    pltpu.sync_copy(ones_hbm.at[pl.ds(0, 128)], ones_v)
    pltpu.sync_copy(ones_hbm.at[pl.ds(sid * RT, RT)], acc.at[pl.ds(sid * RT, RT)])
    plsc.subcore_barrier()

    def body(j, carry):
        pltpu.sync_copy(ones_v, acc.at[dst_v.at[j]], add=True)
        return carry

    lax.fori_loop(0, DB, body, 0)
    plsc.subcore_barrier()
    pltpu.sync_copy(acc.at[pl.ds(sid * RT, RT)], out_hbm.at[cid, pl.ds(sid * RT, RT)])


def _make_scatter(nch):
    rpc = nch // 2

    @functools.partial(
        pl.kernel,
        mesh=plsc.VectorSubcoreMesh(**_MESH),
        out_type=jax.ShapeDtypeStruct((nch, N_PAD, CW), jnp.float32),
        scratch_types=[
            pltpu.VMEM((NB // 2, 128), jnp.int32),
            pltpu.VMEM((NB // 2, 128), jnp.int32),
            pltpu.VMEM((2, 128, CW), jnp.float32),
            pltpu.VMEM_SHARED((N_PAD, CW), jnp.float32),
            pltpu.SemaphoreType.DMA,
            pltpu.SemaphoreType.DMA,
        ],
    )
    def scat(g_hbm, src_hbm, dst_hbm, out_hbm, src_v, dst_v, buf, acc, s0, s1):
        cid = lax.axis_index("c")
        sid = lax.axis_index("s")
        NBH = NB // 2
        for rep in range(rpc):
            chunk = cid * rpc + rep
            pltpu.sync_copy(
                g_hbm.at[pl.ds(chunk * N_PAD + sid * RT, RT)],
                acc.at[pl.ds(sid * RT, RT)],
            )
            plsc.subcore_barrier()

            for half in range(2):
                pltpu.sync_copy(
                    src_hbm.at[chunk, sid, pl.ds(half * NBH, NBH)], src_v)
                pltpu.sync_copy(
                    dst_hbm.at[sid, pl.ds(half * NBH, NBH)], dst_v)

                pltpu.async_copy(g_hbm.at[src_v.at[0]], buf.at[0], s0)

                def body(h, carry):
                    j = 2 * h
                    pltpu.make_async_copy(
                        g_hbm.at[src_v.at[0]], buf.at[0], s0).wait()
                    pltpu.async_copy(g_hbm.at[src_v.at[j + 1]], buf.at[1], s1)
                    pltpu.sync_copy(buf.at[0], acc.at[dst_v.at[j]], add=True)
                    pltpu.make_async_copy(
                        g_hbm.at[src_v.at[0]], buf.at[1], s1).wait()
                    pltpu.async_copy(
                        g_hbm.at[src_v.at[lax.rem(j + 2, NBH)]], buf.at[0], s0)
                    pltpu.sync_copy(buf.at[1], acc.at[dst_v.at[j + 1]], add=True)
                    return carry

                lax.fori_loop(0, NBH // 2, body, 0)
                pltpu.make_async_copy(g_hbm.at[src_v.at[0]], buf.at[0], s0).wait()
            plsc.subcore_barrier()
            pltpu.sync_copy(
                acc.at[pl.ds(sid * RT, RT)],
                out_hbm.at[chunk, pl.ds(sid * RT, RT)],
            )
            if rep + 1 < rpc:
                plsc.subcore_barrier()

    return scat


_scatter2 = _make_scatter(2)


def _dot(a, b):
    return lax.dot_general(a, b, (((1,), (0,)), ((), ())),
                           preferred_element_type=jnp.float32)


def _scale_body(x_ref, d_ref, o_ref):
    g = x_ref[...] * d_ref[...]
    o_ref[0] = g[:, :CW]
    o_ref[1] = g[:, CW:]


def _scale(x_pad, dvec):
    return pl.pallas_call(
        _scale_body,
        grid=(N_PAD // RBLK,),
        in_specs=[
            pl.BlockSpec((RBLK, 256), lambda i: (i, 0)),
            pl.BlockSpec((RBLK, 1), lambda i: (i, 0)),
        ],
        out_specs=pl.BlockSpec((2, RBLK, CW), lambda i: (0, i, 0)),
        out_shape=jax.ShapeDtypeStruct((2, N_PAD, CW), jnp.float32),
    )(x_pad, dvec)


def _mid_body(s_ref, w1_ref, w2_ref, b_ref, d_ref, o_ref):
    t = jnp.concatenate([s_ref[0], s_ref[1]], axis=1)
    h = jnp.maximum(_dot(t, w1_ref[...]) * d_ref[...] + b_ref[...], 0.0)
    g = _dot(h, w2_ref[...]) * d_ref[...]
    o_ref[0] = g[:, :CW]
    o_ref[1] = g[:, CW:]


def _mid(s1, W1, W2, b1, dvec):
    return pl.pallas_call(
        _mid_body,
        grid=(N_PAD // RBLK,),
        in_specs=[
            pl.BlockSpec((2, RBLK, CW), lambda i: (0, i, 0)),
            pl.BlockSpec((256, 512), lambda i: (0, 0)),
            pl.BlockSpec((512, 256), lambda i: (0, 0)),
            pl.BlockSpec((1, 512), lambda i: (0, 0)),
            pl.BlockSpec((RBLK, 1), lambda i: (i, 0)),
        ],
        out_specs=pl.BlockSpec((2, RBLK, CW), lambda i: (0, i, 0)),
        out_shape=jax.ShapeDtypeStruct((2, N_PAD, CW), jnp.float32),
    )(s1, W1, W2, b1, dvec)


def _fin_body(s_ref, b_ref, d_ref, o_ref):
    t = jnp.concatenate([s_ref[0], s_ref[1]], axis=1)
    o_ref[...] = t * d_ref[...] + b_ref[...]


def _fin(s2, b2, dvec):
    return pl.pallas_call(
        _fin_body,
        grid=(N_PAD // RBLK,),
        in_specs=[
            pl.BlockSpec((2, RBLK, CW), lambda i: (0, i, 0)),
            pl.BlockSpec((1, 256), lambda i: (0, 0)),
            pl.BlockSpec((RBLK, 1), lambda i: (i, 0)),
        ],
        out_specs=pl.BlockSpec((RBLK, 256), lambda i: (i, 0)),
        out_shape=jax.ShapeDtypeStruct((N_PAD, 256), jnp.float32),
    )(s2, b2, dvec)


def kernel(x, edge_index, W1, b1, W2, b2):
    src = edge_index[0].astype(jnp.int32)
    dst = edge_index[1].astype(jnp.int32)
    pad = N + (jnp.arange(E_PAD - E, dtype=jnp.int32) % (N_PAD - N))
    src_p = jnp.concatenate([src, pad]).reshape(16, EB)
    dst_t = jnp.concatenate([dst, pad]).reshape(16, NB, 128)
    src2 = (src_p[None] + (jnp.arange(2, dtype=jnp.int32) * N_PAD)[:, None, None]
            ).reshape(2, 16, NB, 128)

    ones_pl = jnp.ones((N_PAD, 128), jnp.float32)
    hist2 = _deg_kernel(dst_t.reshape(16, 2, DB, 128), ones_pl)
    hist = hist2[0, :, 0] + hist2[1, :, 0] - 2.0
    dvec = jnp.where(jnp.arange(N_PAD) < N,
                     lax.rsqrt(hist + 1.0), 0.0).astype(jnp.float32)[:, None]

    x_pad = jnp.pad(x, ((0, N_PAD - N), (0, 0)))
    xs = _scale(x_pad, dvec)
    s1 = _scatter2(xs.reshape(2 * N_PAD, CW), src2, dst_t)
    g2 = _mid(s1, W1, W2, b1.reshape(1, 512), dvec)
    s2 = _scatter2(g2.reshape(2 * N_PAD, CW), src2, dst_t)
    out = _fin(s2, b2.reshape(1, 256), dvec)
    return out[:N]


# speedup vs baseline: 18.8370x; 1.0321x over previous
import functools

import jax
import jax.numpy as jnp
from jax import lax
from jax.experimental import pallas as pl
from jax.experimental.pallas import tpu as pltpu
from jax.experimental.pallas import tpu_sc as plsc

N = 10000
N_PAD = 10240
E = 160000
E_PAD = 163840
NB = 80
EB = E_PAD // 16
CW = 128
RT = N_PAD // 16
DB = NB // 2
RBLK = 1024

_MESH = dict(core_axis_name="c", subcore_axis_name="s")


@functools.partial(
    pl.kernel,
    mesh=plsc.VectorSubcoreMesh(**_MESH),
    out_type=jax.ShapeDtypeStruct((2, N_PAD, 128), jnp.float32),
    scratch_types=[
        pltpu.VMEM((DB, 128), jnp.int32),
        pltpu.VMEM((128, 128), jnp.float32),
        pltpu.VMEM_SHARED((N_PAD, 128), jnp.float32),
    ],
)
def _deg_kernel(dst_hbm, ones_hbm, out_hbm, dst_v, ones_v, acc):
    cid = lax.axis_index("c")
    sid = lax.axis_index("s")
    pltpu.sync_copy(dst_hbm.at[sid, cid], dst_v)
    pltpu.sync_copy(ones_hbm.at[pl.ds(0, 128)], ones_v)
    pltpu.sync_copy(ones_hbm.at[pl.ds(sid * RT, RT)], acc.at[pl.ds(sid * RT, RT)])
    plsc.subcore_barrier()

    def body(j, carry):
        pltpu.sync_copy(ones_v, acc.at[dst_v.at[j]], add=True)
        return carry

    lax.fori_loop(0, DB, body, 0)
    plsc.subcore_barrier()
    pltpu.sync_copy(acc.at[pl.ds(sid * RT, RT)], out_hbm.at[cid, pl.ds(sid * RT, RT)])


def _make_scatter(nch):
    rpc = nch // 2

    @functools.partial(
        pl.kernel,
        mesh=plsc.VectorSubcoreMesh(**_MESH),
        out_type=jax.ShapeDtypeStruct((nch, N_PAD, CW), jnp.float32),
        scratch_types=[
            pltpu.VMEM((NB // 2, 128), jnp.int32),
            pltpu.VMEM((NB // 2, 128), jnp.int32),
            pltpu.VMEM((2, 128, CW), jnp.float32),
            pltpu.VMEM_SHARED((N_PAD, CW), jnp.float32),
            pltpu.SemaphoreType.DMA,
            pltpu.SemaphoreType.DMA,
        ],
    )
    def scat(g_hbm, src_hbm, dst_hbm, out_hbm, src_v, dst_v, buf, acc, s0, s1):
        cid = lax.axis_index("c")
        sid = lax.axis_index("s")
        NBH = NB // 2
        for rep in range(rpc):
            chunk = cid * rpc + rep
            pltpu.sync_copy(
                g_hbm.at[pl.ds(chunk * N_PAD + sid * RT, RT)],
                acc.at[pl.ds(sid * RT, RT)],
            )
            plsc.subcore_barrier()

            for half in range(2):
                pltpu.sync_copy(
                    src_hbm.at[chunk, sid, pl.ds(half * NBH, NBH)], src_v)
                pltpu.sync_copy(
                    dst_hbm.at[sid, pl.ds(half * NBH, NBH)], dst_v)

                pltpu.async_copy(g_hbm.at[src_v.at[0]], buf.at[0], s0)

                def body(h, carry):
                    j = 2 * h
                    pltpu.make_async_copy(
                        g_hbm.at[src_v.at[0]], buf.at[0], s0).wait()
                    pltpu.async_copy(g_hbm.at[src_v.at[j + 1]], buf.at[1], s1)
                    pltpu.sync_copy(buf.at[0], acc.at[dst_v.at[j]], add=True)
                    pltpu.make_async_copy(
                        g_hbm.at[src_v.at[0]], buf.at[1], s1).wait()
                    pltpu.async_copy(
                        g_hbm.at[src_v.at[lax.rem(j + 2, NBH)]], buf.at[0], s0)
                    pltpu.sync_copy(buf.at[1], acc.at[dst_v.at[j + 1]], add=True)
                    return carry

                lax.fori_loop(0, NBH // 2, body, 0)
                pltpu.make_async_copy(g_hbm.at[src_v.at[0]], buf.at[0], s0).wait()
            plsc.subcore_barrier()
            pltpu.sync_copy(
                acc.at[pl.ds(sid * RT, RT)],
                out_hbm.at[chunk, pl.ds(sid * RT, RT)],
            )
            if rep + 1 < rpc:
                plsc.subcore_barrier()

    return scat


_scatter2 = _make_scatter(2)


def _dot(a, b):
    return lax.dot_general(a, b, (((1,), (0,)), ((), ())),
                           preferred_element_type=jnp.float32)


def _scale_body(x_ref, h_ref, o_ref, d_ref):
    row0 = pl.program_id(0) * RBLK
    rows = row0 + lax.broadcasted_iota(jnp.int32, (RBLK, 1), 0)
    h = h_ref[0][:, :1] + h_ref[1][:, :1] - 1.0
    d = jnp.where(rows < N, lax.rsqrt(h), 0.0)
    d_ref[...] = d
    g = x_ref[...] * d
    o_ref[0] = g[:, :CW]
    o_ref[1] = g[:, CW:]


def _scale(x_pad, hist2):
    return pl.pallas_call(
        _scale_body,
        grid=(N_PAD // RBLK,),
        in_specs=[
            pl.BlockSpec((RBLK, 256), lambda i: (i, 0)),
            pl.BlockSpec((2, RBLK, 128), lambda i: (0, i, 0)),
        ],
        out_specs=[
            pl.BlockSpec((2, RBLK, CW), lambda i: (0, i, 0)),
            pl.BlockSpec((RBLK, 1), lambda i: (i, 0)),
        ],
        out_shape=[
            jax.ShapeDtypeStruct((2, N_PAD, CW), jnp.float32),
            jax.ShapeDtypeStruct((N_PAD, 1), jnp.float32),
        ],
    )(x_pad, hist2)


def _mid_body(s_ref, w1_ref, w2_ref, b_ref, d_ref, o_ref):
    t = jnp.concatenate([s_ref[0], s_ref[1]], axis=1)
    h = jnp.maximum(_dot(t, w1_ref[...]) * d_ref[...] + b_ref[...], 0.0)
    g = _dot(h, w2_ref[...]) * d_ref[...]
    o_ref[0] = g[:, :CW]
    o_ref[1] = g[:, CW:]


def _mid(s1, W1, W2, b1, dvec):
    return pl.pallas_call(
        _mid_body,
        grid=(N_PAD // RBLK,),
        in_specs=[
            pl.BlockSpec((2, RBLK, CW), lambda i: (0, i, 0)),
            pl.BlockSpec((256, 512), lambda i: (0, 0)),
            pl.BlockSpec((512, 256), lambda i: (0, 0)),
            pl.BlockSpec((1, 512), lambda i: (0, 0)),
            pl.BlockSpec((RBLK, 1), lambda i: (i, 0)),
        ],
        out_specs=pl.BlockSpec((2, RBLK, CW), lambda i: (0, i, 0)),
        out_shape=jax.ShapeDtypeStruct((2, N_PAD, CW), jnp.float32),
    )(s1, W1, W2, b1, dvec)


def _fin_body(s_ref, b_ref, d_ref, o_ref):
    t = jnp.concatenate([s_ref[0], s_ref[1]], axis=1)
    o_ref[...] = t * d_ref[...] + b_ref[...]


def _fin(s2, b2, dvec):
    return pl.pallas_call(
        _fin_body,
        grid=(N_PAD // RBLK,),
        in_specs=[
            pl.BlockSpec((2, RBLK, CW), lambda i: (0, i, 0)),
            pl.BlockSpec((1, 256), lambda i: (0, 0)),
            pl.BlockSpec((RBLK, 1), lambda i: (i, 0)),
        ],
        out_specs=pl.BlockSpec((RBLK, 256), lambda i: (i, 0)),
        out_shape=jax.ShapeDtypeStruct((N, 256), jnp.float32),
    )(s2, b2, dvec)


def kernel(x, edge_index, W1, b1, W2, b2):
    src = edge_index[0].astype(jnp.int32)
    dst = edge_index[1].astype(jnp.int32)
    pad = N + (jnp.arange(E_PAD - E, dtype=jnp.int32) % (N_PAD - N))
    src_p = jnp.concatenate([src, pad]).reshape(16, EB)
    dst_t = jnp.concatenate([dst, pad]).reshape(16, NB, 128)
    src2 = (src_p[None] + (jnp.arange(2, dtype=jnp.int32) * N_PAD)[:, None, None]
            ).reshape(2, 16, NB, 128)

    ones_pl = jnp.ones((N_PAD, 128), jnp.float32)
    hist2 = _deg_kernel(dst_t.reshape(16, 2, DB, 128), ones_pl)

    x_pad = jnp.pad(x, ((0, N_PAD - N), (0, 0)))
    xs, dvec = _scale(x_pad, hist2)
    s1 = _scatter2(xs.reshape(2 * N_PAD, CW), src2, dst_t)
    g2 = _mid(s1, W1, W2, b1.reshape(1, 512), dvec)
    s2 = _scatter2(g2.reshape(2 * N_PAD, CW), src2, dst_t)
    return _fin(s2, b2.reshape(1, 256), dvec)
